# baseline (speedup 1.0000x reference)
import functools

import jax
import jax.numpy as jnp
from jax import lax
from jax.experimental import pallas as pl
from jax.experimental.pallas import tpu as pltpu
from jax.experimental.pallas import tpu_sc as plsc

_D = 64
_DW = _D // 2
_TK = 200 * _D
_TKW = 200 * _DW
_NB = 4096
_S = 4
_SB = _NB // _S
_SROWS = _SB * 200

_info = plsc.get_sparse_core_info()
_NC = _info.num_cores
_NS = _info.num_subcores
_NW = _NC * _NS
_ROWS_PER_W = _SROWS // _NW
_CHUNK = 800
_N_CHUNKS = _ROWS_PER_W // _CHUNK

_mesh = plsc.VectorSubcoreMesh(core_axis_name="c", subcore_axis_name="s")


def _make_gather(slab):
    @functools.partial(
        pl.kernel,
        mesh=_mesh,
        out_type=jax.ShapeDtypeStruct((_SROWS, _DW), jnp.int32),
        compiler_params=pltpu.CompilerParams(use_tc_tiling_on_sc=False),
        scratch_types=[
            pltpu.VMEM((_CHUNK,), jnp.int32),
            pltpu.VMEM((_CHUNK, _DW), jnp.int32),
            pltpu.SemaphoreType.DMA,
        ],
    )
    def _gather(idx_hbm, table_hbm, out_hbm, idx_v, rows_v, sem):
        wid = lax.axis_index("s") * _NC + lax.axis_index("c")
        base = pl.multiple_of(wid * _ROWS_PER_W, 8)
        src_base = pl.multiple_of(slab * _SROWS + wid * _ROWS_PER_W, 8)

        def body(i, carry):
            off = pl.multiple_of(base + i * _CHUNK, 8)
            src = pl.multiple_of(src_base + i * _CHUNK, 8)
            pltpu.sync_copy(idx_hbm.at[pl.ds(src, _CHUNK)], idx_v)
            pltpu.async_copy(table_hbm.at[idx_v], rows_v, sem).wait()
            pltpu.sync_copy(rows_v, out_hbm.at[pl.ds(off, _CHUNK)])
            return carry

        lax.fori_loop(0, _N_CHUNKS, body, 0)

    return _gather


_gathers = [_make_gather(s) for s in range(_S)]

_BB = 128
_SSTEPS = _SB // _BB


def _tr_body(slab, lin_hbm, *rest):
    if len(rest) == 4:
        _, o_ref, buf, sems = rest
    else:
        o_ref, buf, sems = rest
    step = pl.program_id(0)
    slot = lax.rem(step, 2)
    nslot = lax.rem(step + 1, 2)

    def issue(dst_slot, blk):
        for r in range(_BB):
            pltpu.make_async_copy(
                lin_hbm.at[pl.ds((blk * _BB + r) * _TKW, _TKW)],
                buf.at[dst_slot, r],
                sems.at[dst_slot],
            ).start()

    def drain(dst_slot, blk):
        for r in range(_BB):
            pltpu.make_async_copy(
                lin_hbm.at[pl.ds((blk * _BB + r) * _TKW, _TKW)],
                buf.at[dst_slot, r],
                sems.at[dst_slot],
            ).wait()

    @pl.when(step == 0)
    def _():
        issue(slot, step)

    @pl.when(step + 1 < _SSTEPS)
    def _():
        issue(nslot, step + 1)

    drain(slot, step)
    for j in range(_TKW // 128):
        w = buf[slot, :, pl.ds(j * 128, 128)]
        y_lo = lax.bitcast_convert_type(w << 16, jnp.float32).T
        y_hi = lax.bitcast_convert_type(w & jnp.int32(-65536), jnp.float32).T
        for t_in in range(4):
            t_abs = 4 * j + t_in
            o_ref[pl.ds(t_abs * _D, 32), :] = y_lo[t_in * 32:(t_in + 1) * 32, :]
            o_ref[pl.ds(t_abs * _D + 32, 32), :] = (
                y_hi[t_in * 32:(t_in + 1) * 32, :])


def _make_transpose(slab, aliased):
    in_specs = [pl.BlockSpec(memory_space=pl.ANY)]
    if aliased:
        in_specs.append(pl.BlockSpec(memory_space=pl.ANY))
    return pl.pallas_call(
        functools.partial(_tr_body, slab),
        grid=(_SSTEPS,),
        in_specs=in_specs,
        out_specs=pl.BlockSpec((_TK, _BB), lambda i, s=slab: (0, s * _SSTEPS + i)),
        out_shape=jax.ShapeDtypeStruct((_TK, _NB), jnp.float32),
        input_output_aliases={1: 0} if aliased else {},
        scratch_shapes=[
            pltpu.VMEM((2, _BB, _TKW), jnp.int32),
            pltpu.SemaphoreType.DMA((2,)),
        ],
    )


_transposes = [_make_transpose(s, s > 0) for s in range(_S)]


def kernel(idxes, pe):
    flat = idxes.reshape(-1).astype(jnp.int32)
    u = lax.bitcast_convert_type(pe, jnp.uint32)
    r = (u + jnp.uint32(0x7FFF) + ((u >> 16) & jnp.uint32(1))) >> 16
    pe32 = lax.bitcast_convert_type(r[:, :_DW] | (r[:, _DW:] << 16), jnp.int32)
    lins = [_gathers[s](flat, pe32) for s in range(_S)]
    acc = _transposes[0](lins[0].reshape(-1))
    for s in range(1, _S):
        acc = _transposes[s](lins[s].reshape(-1), acc)
    return acc.reshape(200, _D, _NB).transpose(2, 0, 1)


# speedup vs baseline: 1.1408x; 1.1408x over previous
import functools

import jax
import jax.numpy as jnp
from jax import lax
from jax.experimental import pallas as pl
from jax.experimental.pallas import tpu as pltpu
from jax.experimental.pallas import tpu_sc as plsc

_D = 64
_DW = _D // 2
_TK = 200 * _D
_TKW = 200 * _DW
_NB = 4096
_S = 4
_SB = _NB // _S
_SROWS = _SB * 200

_info = plsc.get_sparse_core_info()
_NC = _info.num_cores
_NS = _info.num_subcores
_NW = _NC * _NS
_ROWS_PER_W = _SROWS // _NW
_CHUNK = 800
_N_CHUNKS = _ROWS_PER_W // _CHUNK

_mesh = plsc.VectorSubcoreMesh(core_axis_name="c", subcore_axis_name="s")


def _make_gather(slab):
    @functools.partial(
        pl.kernel,
        mesh=_mesh,
        out_type=jax.ShapeDtypeStruct((_SROWS, _DW), jnp.int32),
        compiler_params=pltpu.CompilerParams(use_tc_tiling_on_sc=False),
        scratch_types=[
            pltpu.VMEM((_CHUNK,), jnp.int32),
            pltpu.VMEM((_CHUNK, _DW), jnp.int32),
            pltpu.SemaphoreType.DMA,
        ],
    )
    def _gather(idx_hbm, table_hbm, out_hbm, idx_v, rows_v, sem):
        wid = lax.axis_index("s") * _NC + lax.axis_index("c")
        base = pl.multiple_of(wid * _ROWS_PER_W, 8)
        src_base = pl.multiple_of(slab * _SROWS + wid * _ROWS_PER_W, 8)

        def body(i, carry):
            off = pl.multiple_of(base + i * _CHUNK, 8)
            src = pl.multiple_of(src_base + i * _CHUNK, 8)
            pltpu.sync_copy(idx_hbm.at[pl.ds(src, _CHUNK)], idx_v)
            pltpu.async_copy(table_hbm.at[idx_v], rows_v, sem).wait()
            pltpu.sync_copy(rows_v, out_hbm.at[pl.ds(off, _CHUNK)])
            return carry

        lax.fori_loop(0, _N_CHUNKS, body, 0)

    return _gather


_gathers = [_make_gather(s) for s in range(_S)]

_BB = 128
_SSTEPS = _SB // _BB


def _tr_body(slab, lin_hbm, *rest):
    if len(rest) == 4:
        _, o_ref, buf, sems = rest
    else:
        o_ref, buf, sems = rest
    step = pl.program_id(0)
    slot = lax.rem(step, 2)
    nslot = lax.rem(step + 1, 2)

    def issue(dst_slot, blk):
        for r in range(_BB):
            pltpu.make_async_copy(
                lin_hbm.at[pl.ds((blk * _BB + r) * _TKW, _TKW)],
                buf.at[dst_slot, r],
                sems.at[dst_slot],
            ).start()

    def drain(dst_slot, blk):
        for r in range(_BB):
            pltpu.make_async_copy(
                lin_hbm.at[pl.ds((blk * _BB + r) * _TKW, _TKW)],
                buf.at[dst_slot, r],
                sems.at[dst_slot],
            ).wait()

    @pl.when(step == 0)
    def _():
        issue(slot, step)

    @pl.when(step + 1 < _SSTEPS)
    def _():
        issue(nslot, step + 1)

    drain(slot, step)
    for j in range(_TKW // 128):
        w = buf[slot, :, pl.ds(j * 128, 128)]
        y_lo = lax.bitcast_convert_type(w << 16, jnp.float32).T
        y_hi = lax.bitcast_convert_type(w & jnp.int32(-65536), jnp.float32).T
        for t_in in range(4):
            t_abs = 4 * j + t_in
            o_ref[pl.ds(t_abs * _D, 32), :] = y_lo[t_in * 32:(t_in + 1) * 32, :]
            o_ref[pl.ds(t_abs * _D + 32, 32), :] = (
                y_hi[t_in * 32:(t_in + 1) * 32, :])


def _make_transpose(slab, aliased):
    in_specs = [pl.BlockSpec(memory_space=pl.ANY)]
    if aliased:
        in_specs.append(pl.BlockSpec(memory_space=pl.ANY))
    return pl.pallas_call(
        functools.partial(_tr_body, slab),
        grid=(_SSTEPS,),
        in_specs=in_specs,
        out_specs=pl.BlockSpec((_TK, _BB), lambda i, s=slab: (0, s * _SSTEPS + i)),
        out_shape=jax.ShapeDtypeStruct((_TK, _NB), jnp.float32),
        input_output_aliases={1: 0} if aliased else {},
        scratch_shapes=[
            pltpu.VMEM((2, _BB, _TKW), jnp.int32),
            pltpu.SemaphoreType.DMA((2,)),
        ],
    )


_transposes = [_make_transpose(s, s > 0) for s in range(_S)]


def kernel(idxes, pe):
    flat = idxes.reshape(-1).astype(jnp.int32)
    pe_bf = pe.astype(jnp.bfloat16)
    pe_pair = jnp.stack([pe_bf[:, :_DW], pe_bf[:, _DW:]], axis=-1)
    pe32 = lax.bitcast_convert_type(pe_pair, jnp.int32)
    lins = [_gathers[s](flat, pe32) for s in range(_S)]
    acc = _transposes[0](lins[0].reshape(-1))
    for s in range(1, _S):
        acc = _transposes[s](lins[s].reshape(-1), acc)
    return acc.reshape(200, _D, _NB).transpose(2, 0, 1)
